# baseline (speedup 1.0000x reference)
import functools

import jax
import jax.numpy as jnp
from jax import lax
from jax.experimental import pallas as pl
from jax.experimental.pallas import tpu as pltpu
from jax.experimental.pallas import tpu_sc as plsc

_NC = 2
_NS = 16
_NW = _NC * _NS


def _sc_pair_gather(ids2, x, chunk):
    n = ids2.shape[0]
    d = x.shape[1]
    per_w = n // _NW
    n_chunks = per_w // chunk
    mesh = plsc.VectorSubcoreMesh(core_axis_name="c", subcore_axis_name="s")

    @functools.partial(
        pl.kernel,
        out_type=jax.ShapeDtypeStruct((n, d), jnp.float32),
        mesh=mesh,
        scratch_types=[
            pltpu.VMEM((chunk,), jnp.int32),
            pltpu.VMEM((chunk, d), jnp.float32),
            pltpu.SemaphoreType.DMA,
        ],
    )
    def k(ids_hbm, x_hbm, out_hbm, idx_v, rows_v, sem):
        wid = lax.axis_index("s") * _NC + lax.axis_index("c")
        base = wid * per_w

        def body(i, carry):
            off = base + i * chunk
            pltpu.sync_copy(ids_hbm.at[pl.ds(off, chunk)], idx_v)
            pltpu.async_copy(x_hbm.at[idx_v], rows_v, sem).wait()
            pltpu.sync_copy(rows_v, out_hbm.at[pl.ds(off, chunk)])
            return carry

        lax.fori_loop(0, n_chunks, body, 0)

    return k(ids2, x)


def _tc_epilogue(rows, par, pos_rep, gamma, beta, eps=1e-5):
    n, d2 = rows.shape
    e = d2 // 2
    bb = pos_rep.shape[0]

    def body(r_ref, p_ref, pos_ref, g_ref, b_ref, o_ref):
        r = r_ref[...]
        p = p_ref[...]
        x = jnp.where(p == 1.0, r[:, e:], r[:, :e]) + pos_ref[...]
        ones = jnp.ones((e, 1), jnp.float32)
        mean = lax.dot_general(x, ones, (((1,), (0,)), ((), ()))) * (1.0 / e)
        xc = x - mean
        var = lax.dot_general(xc * xc, ones, (((1,), (0,)), ((), ()))) * (1.0 / e)
        o_ref[...] = xc * (lax.rsqrt(var + eps) * g_ref[...]) + b_ref[...]

    return pl.pallas_call(
        body,
        grid=(n // bb,),
        in_specs=[
            pl.BlockSpec((bb, d2), lambda i: (i, 0)),
            pl.BlockSpec((bb, 1), lambda i: (i, 0)),
            pl.BlockSpec((bb, e), lambda i: (0, 0)),
            pl.BlockSpec((1, e), lambda i: (0, 0)),
            pl.BlockSpec((1, e), lambda i: (0, 0)),
        ],
        out_specs=pl.BlockSpec((bb, e), lambda i: (i, 0)),
        out_shape=jax.ShapeDtypeStruct((n, e), jnp.float32),
    )(rows, par, pos_rep, gamma.reshape(1, e), beta.reshape(1, e))


def kernel(input_ids, tok_table, pos_table, ln_gamma, ln_beta):
    b, l = input_ids.shape
    e = tok_table.shape[1]
    n = b * l
    ids = input_ids.astype(jnp.int32).reshape(-1)
    x = tok_table.reshape(tok_table.shape[0] // 2, 2 * e)
    rows = _sc_pair_gather(ids >> 1, x, chunk=640)
    par = (ids & 1).astype(jnp.float32).reshape(n, 1)
    bb = 16 * l
    pos_rep = jnp.tile(pos_table[:l], (bb // l, 1))
    out = _tc_epilogue(rows, par, pos_rep, ln_gamma, ln_beta)
    return out.reshape(b, l, e)
